# baseline (speedup 1.0000x reference)
import functools

import jax
import jax.numpy as jnp
from jax import lax
from jax.experimental import pallas as pl
from jax.experimental.pallas import tpu as pltpu
from jax.experimental.pallas import tpu_sc as plsc

NC = 2
NS = 16
NW = NC * NS
NBUF = 16
BCH = 64
L = 16


@jax.jit
def _gather_sc(ids, weight):
    B, F = ids.shape
    D = weight.shape[1]
    FD = F * D
    rows_per_w = B // NW
    n_outer = rows_per_w // NBUF
    t_per_blk = BCH // NBUF
    n_win = FD // L
    mesh = plsc.VectorSubcoreMesh(core_axis_name="c", subcore_axis_name="s")

    @functools.partial(
        pl.kernel,
        mesh=mesh,
        compiler_params=pltpu.CompilerParams(
            use_tc_tiling_on_sc=False, needs_layout_passes=False
        ),
        out_type=jax.ShapeDtypeStruct((FD, B), jnp.float32),
        scratch_types=[
            pltpu.VMEM((rows_per_w, F), jnp.int32),
            pltpu.VMEM((NBUF, F, D), jnp.float32),
            pltpu.VMEM((FD, BCH), jnp.float32),
            pltpu.VMEM((n_win, L), jnp.int32),
            pltpu.SemaphoreType.DMA,
            pltpu.SemaphoreType.DMA,
        ],
    )
    def k(idx_hbm, table_hbm, out_hbm, idx_v, rows_v, blk_v, tab_v, gsem, bsem):
        wid = lax.axis_index("s") * NC + lax.axis_index("c")
        row0 = wid * rows_per_w
        pltpu.sync_copy(idx_hbm.at[pl.ds(row0, rows_per_w)], idx_v)

        iota = lax.broadcasted_iota(jnp.int32, (L,), 0)
        for w in range(n_win):
            tab_v[w] = iota + w * L

        def fire_gather(r, slot):
            pltpu.async_copy(table_hbm.at[idx_v.at[r]], rows_v.at[slot], gsem)

        def wait_gather():
            pltpu.make_async_copy(
                out_hbm.at[pl.ds(0, F), pl.ds(0, D)], rows_v.at[0], gsem
            ).wait()

        def wait_block():
            pltpu.make_async_copy(
                blk_v, out_hbm.at[:, pl.ds(0, BCH)], bsem
            ).wait()

        for u in range(NBUF):
            fire_gather(u, u)

        def body(t, carry):
            @pl.when((t % t_per_blk == 0) & (t >= t_per_blk))
            def _():
                wait_block()

            jbase = (t % t_per_blk) * NBUF
            for u in range(NBUF):
                r = t * NBUF + u
                wait_gather()
                b_idx = jnp.full((L,), jbase + u, jnp.int32)
                for w in range(n_win):
                    f = w // (D // L)
                    h = w % (D // L)
                    v = rows_v[u, f, pl.ds(h * L, L)]
                    plsc.store_scatter(blk_v, [tab_v[w], b_idx], v)

                @pl.when(r + NBUF < rows_per_w)
                def _():
                    fire_gather(r + NBUF, u)

            @pl.when(t % t_per_blk == t_per_blk - 1)
            def _():
                pltpu.async_copy(
                    blk_v,
                    out_hbm.at[:, pl.ds(row0 + (t // t_per_blk) * BCH, BCH)],
                    bsem,
                )

            return carry

        lax.fori_loop(0, n_outer, body, 0)
        wait_block()

    out_fm = k(ids, weight)
    return jnp.transpose(out_fm.reshape(F, D, B), (2, 0, 1))


def kernel(token_ids, weight):
    return _gather_sc(token_ids.astype(jnp.int32), weight)


# speedup vs baseline: 1.2371x; 1.2371x over previous
import functools

import jax
import jax.numpy as jnp
from jax import lax
from jax.experimental import pallas as pl
from jax.experimental.pallas import tpu as pltpu
from jax.experimental.pallas import tpu_sc as plsc

NC = 2
NS = 16
NW = NC * NS
CHUNK = 1664


@functools.partial(jax.jit, static_argnums=(2, 3))
def _gather_sc(idx3, weight, n_per_w, n_chunks):
    D = weight.shape[1]
    N = NW * n_per_w
    mesh = plsc.VectorSubcoreMesh(core_axis_name="c", subcore_axis_name="s")

    @functools.partial(
        pl.kernel,
        mesh=mesh,
        compiler_params=pltpu.CompilerParams(use_tc_tiling_on_sc=False),
        out_type=jax.ShapeDtypeStruct((N, D), jnp.float32),
        scratch_types=[
            pltpu.VMEM((n_chunks, CHUNK), jnp.int32),
            pltpu.VMEM((2, CHUNK, D), jnp.float32),
            pltpu.SemaphoreType.DMA,
            pltpu.SemaphoreType.DMA,
        ],
    )
    def k(idx_hbm, table_hbm, out_hbm, idx_v, rows_v, gsem, wsem):
        wid = lax.axis_index("s") * NC + lax.axis_index("c")
        base = wid * n_per_w
        pltpu.sync_copy(idx_hbm.at[wid], idx_v)

        gcp = [None] * n_chunks
        wcp = [None] * n_chunks
        gcp[0] = pltpu.async_copy(table_hbm.at[idx_v.at[0]], rows_v.at[0], gsem)
        for j in range(n_chunks):
            nb = j + 1
            if nb < n_chunks:
                if nb >= 2:
                    wcp[nb - 2].wait()
                gcp[nb] = pltpu.async_copy(
                    table_hbm.at[idx_v.at[nb]], rows_v.at[nb % 2], gsem
                )
            gcp[j].wait()
            wcp[j] = pltpu.async_copy(
                rows_v.at[j % 2], out_hbm.at[pl.ds(base + j * CHUNK, CHUNK)], wsem
            )
        wcp[n_chunks - 2].wait()
        wcp[n_chunks - 1].wait()

    return k(idx3, weight)


def kernel(token_ids, weight):
    B, F = token_ids.shape
    N = B * F
    assert N % (NW * CHUNK) == 0
    n_per_w = N // NW
    n_chunks = n_per_w // CHUNK
    idx3 = token_ids.astype(jnp.int32).reshape(NW, n_chunks, CHUNK)
    out = _gather_sc(idx3, weight, n_per_w, n_chunks)
    return out.reshape(B, F, weight.shape[1])
